# baseline (speedup 1.0000x reference)
import functools

import jax
import jax.numpy as jnp
from jax import lax
from jax.experimental import pallas as pl
from jax.experimental.pallas import tpu as pltpu
from jax.experimental.pallas import tpu_sc as plsc

_N = 10000
_E = 320000
_IN_F = 128
_OUT_F = 128
_EMB = 32

_NC = 2
_NS = 16
_L = 16
_NW = _NC * _NS
_EPW = _E // _NW
_KC = 80
_NCHUNK = _EPW // _KC
_NPAD = 10240
_RPT = _NPAD // _NS

_ROW_BLK = 1000


def _proj_body(belta_ref, feat_ref, w1t_ref, b1_ref, wst_ref, bs_ref,
               wdt_ref, bd_ref, wat_ref, ba_ref,
               fsrc_ref, semb_ref, demb_ref, eatt_ref):
    f = feat_ref[...]
    b = belta_ref[0]
    fsrc_ref[...] = jnp.dot(f, w1t_ref[...],
                            preferred_element_type=jnp.float32) + b1_ref[...]
    semb_ref[...] = (jnp.dot(f, wst_ref[...],
                             preferred_element_type=jnp.float32)
                     + bs_ref[...]) * b
    demb_ref[...] = jnp.dot(f, wdt_ref[...],
                            preferred_element_type=jnp.float32) + bd_ref[...]
    eatt_ref[...] = jnp.dot(jnp.maximum(f, 0.0), wat_ref[...],
                            preferred_element_type=jnp.float32) + ba_ref[...]


def _projections(feat, w1t, b1, wst, bs, wdt, bd, wat, ba, belta):
    nblk = _N // _ROW_BLK
    full = lambda *_: (0, 0)
    row = lambda i: (i, 0)
    return pl.pallas_call(
        _proj_body,
        grid=(nblk,),
        in_specs=[
            pl.BlockSpec(memory_space=pltpu.SMEM),
            pl.BlockSpec((_ROW_BLK, _IN_F), row),
            pl.BlockSpec((_IN_F, _OUT_F), full),
            pl.BlockSpec((1, _OUT_F), full),
            pl.BlockSpec((_IN_F, _EMB), full),
            pl.BlockSpec((1, _EMB), full),
            pl.BlockSpec((_IN_F, _EMB), full),
            pl.BlockSpec((1, _EMB), full),
            pl.BlockSpec((_IN_F, _OUT_F), full),
            pl.BlockSpec((1, _OUT_F), full),
        ],
        out_specs=[
            pl.BlockSpec((_ROW_BLK, _OUT_F), row),
            pl.BlockSpec((_ROW_BLK, _EMB), row),
            pl.BlockSpec((_ROW_BLK, _EMB), row),
            pl.BlockSpec((_ROW_BLK, _OUT_F), row),
        ],
        out_shape=[
            jax.ShapeDtypeStruct((_N, _OUT_F), jnp.float32),
            jax.ShapeDtypeStruct((_N, _EMB), jnp.float32),
            jax.ShapeDtypeStruct((_N, _EMB), jnp.float32),
            jax.ShapeDtypeStruct((_N, _OUT_F), jnp.float32),
        ],
    )(belta, feat, w1t, b1, wst, bs, wdt, bd, wat, ba)


def _edge_body(src_hbm, dst_hbm, dist_hbm, semb_hbm, demb_hbm, fsrc_hbm,
               zeros_hbm, out_hbm,
               si0, si1, di0, di1, dv0, dv1, sr0, sr1, dr0, dr1, fr0, fr1,
               ft_sh, sem_l0, sem_l1, sem_b0, sem_b1):
    cid = lax.axis_index("c")
    sid = lax.axis_index("s")
    wid = sid * _NC + cid
    last = _NCHUNK - 1

    def fire_lin(c, si, di, dv, sem):
        pltpu.async_copy(src_hbm.at[wid, c], si, sem)
        pltpu.async_copy(dst_hbm.at[wid, c], di, sem)
        pltpu.async_copy(dist_hbm.at[wid, c], dv, sem)

    def wait_lin(si, di, dv, sem):
        pltpu.make_async_copy(src_hbm.at[wid, 0], si, sem).wait()
        pltpu.make_async_copy(dst_hbm.at[wid, 0], di, sem).wait()
        pltpu.make_async_copy(dist_hbm.at[wid, 0], dv, sem).wait()

    def fire_gath(si, di, sr, dr, fr, sem):
        pltpu.async_copy(semb_hbm.at[si], sr, sem)
        pltpu.async_copy(demb_hbm.at[di], dr, sem)
        pltpu.async_copy(fsrc_hbm.at[si], fr, sem)

    def wait_gath(si, di, sr, dr, fr, sem):
        pltpu.make_async_copy(semb_hbm.at[si], sr, sem).wait()
        pltpu.make_async_copy(demb_hbm.at[di], dr, sem).wait()
        pltpu.make_async_copy(fsrc_hbm.at[si], fr, sem).wait()

    def compute_scale(dv, sr, dr, fr):
        for g in range(_KC // _L):
            rowi = lax.iota(jnp.int32, _L) + (g * _L)
            acc = jnp.zeros((_L,), jnp.float32)
            for k in range(_EMB):
                colk = jnp.full((_L,), k, jnp.int32)
                s = plsc.load_gather(sr, [rowi, colk])
                d = plsc.load_gather(dr, [rowi, colk])
                acc = acc + s * d
            wvec = acc / dv[pl.ds(g * _L, _L)]
            for i in range(_L):
                e = g * _L + i
                w = jnp.broadcast_to(wvec[i], (_L,))
                for j in range(_OUT_F // _L):
                    sl = pl.ds(j * _L, _L)
                    fr[e, sl] = fr[e, sl] * w

    def scatter(di, fr):
        pltpu.sync_copy(fr, ft_sh.at[di], add=True)

    fire_lin(0, si0, di0, dv0, sem_l0)
    fire_lin(1, si1, di1, dv1, sem_l1)
    pltpu.sync_copy(zeros_hbm, ft_sh.at[pl.ds(sid * _RPT, _RPT)])
    plsc.subcore_barrier()
    wait_lin(si0, di0, dv0, sem_l0)
    fire_gath(si0, di0, sr0, dr0, fr0, sem_b0)

    def pair(p, carry):
        c0 = 2 * p
        wait_gath(si0, di0, sr0, dr0, fr0, sem_b0)
        wait_lin(si1, di1, dv1, sem_l1)
        fire_gath(si1, di1, sr1, dr1, fr1, sem_b1)
        compute_scale(dv0, sr0, dr0, fr0)
        scatter(di0, fr0)
        fire_lin(jnp.minimum(c0 + 2, last), si0, di0, dv0, sem_l0)
        wait_gath(si1, di1, sr1, dr1, fr1, sem_b1)
        wait_lin(si0, di0, dv0, sem_l0)
        fire_gath(si0, di0, sr0, dr0, fr0, sem_b0)
        compute_scale(dv1, sr1, dr1, fr1)
        scatter(di1, fr1)
        fire_lin(jnp.minimum(c0 + 3, last), si1, di1, dv1, sem_l1)
        return carry

    lax.fori_loop(0, _NCHUNK // 2, pair, 0)

    wait_gath(si0, di0, sr0, dr0, fr0, sem_b0)
    compute_scale(dv0, sr0, dr0, fr0)
    scatter(di0, fr0)
    wait_lin(si1, di1, dv1, sem_l1)

    plsc.subcore_barrier()
    off = (cid * _NS + sid) * _RPT
    pltpu.sync_copy(ft_sh.at[pl.ds(sid * _RPT, _RPT)],
                    out_hbm.at[pl.ds(off, _RPT)])


def _edge_aggregate(src, dst, dist, semb, demb, fsrc):
    src3 = src.reshape(_NW, _NCHUNK, _KC)
    dst3 = dst.reshape(_NW, _NCHUNK, _KC)
    dist3 = dist.reshape(_NW, _NCHUNK, _KC)
    zeros = jnp.zeros((_RPT, _OUT_F), jnp.float32)
    mesh = plsc.VectorSubcoreMesh(core_axis_name="c", subcore_axis_name="s")
    run = pl.kernel(
        _edge_body,
        out_type=jax.ShapeDtypeStruct((_NC * _NPAD, _OUT_F), jnp.float32),
        mesh=mesh,
        compiler_params=pltpu.CompilerParams(needs_layout_passes=False,
                                             use_tc_tiling_on_sc=False),
        scratch_types=[
            pltpu.VMEM((_KC,), jnp.int32),
            pltpu.VMEM((_KC,), jnp.int32),
            pltpu.VMEM((_KC,), jnp.int32),
            pltpu.VMEM((_KC,), jnp.int32),
            pltpu.VMEM((_KC,), jnp.float32),
            pltpu.VMEM((_KC,), jnp.float32),
            pltpu.VMEM((_KC, _EMB), jnp.float32),
            pltpu.VMEM((_KC, _EMB), jnp.float32),
            pltpu.VMEM((_KC, _EMB), jnp.float32),
            pltpu.VMEM((_KC, _EMB), jnp.float32),
            pltpu.VMEM((_KC, _OUT_F), jnp.float32),
            pltpu.VMEM((_KC, _OUT_F), jnp.float32),
            pltpu.VMEM_SHARED((_NPAD, _OUT_F), jnp.float32),
            pltpu.SemaphoreType.DMA,
            pltpu.SemaphoreType.DMA,
            pltpu.SemaphoreType.DMA,
            pltpu.SemaphoreType.DMA,
        ],
    )
    return run(src3, dst3, dist3, semb, demb, fsrc, zeros)


def _final_body(eatt_ref, ft_ref, out_ref):
    x = eatt_ref[...] * (ft_ref[0] + ft_ref[1])
    out_ref[...] = jnp.where(x > 0.0, x, jnp.exp(x) - 1.0)


def _finalize(eatt, ft2):
    nblk = _N // _ROW_BLK
    return pl.pallas_call(
        _final_body,
        grid=(nblk,),
        in_specs=[
            pl.BlockSpec((_ROW_BLK, _OUT_F), lambda i: (i, 0)),
            pl.BlockSpec((2, _ROW_BLK, _OUT_F), lambda i: (0, i, 0)),
        ],
        out_specs=pl.BlockSpec((_ROW_BLK, _OUT_F), lambda i: (i, 0)),
        out_shape=jax.ShapeDtypeStruct((_N, _OUT_F), jnp.float32),
    )(eatt, ft2)


def kernel(feat, edge_index, edge_dist, W1, b1, Wsrc, bsrc, Wdst, bdst,
           Watt, batt, belta):
    src = edge_index[0]
    dst = edge_index[1]
    fsrc, semb, demb, eatt = _projections(
        feat, W1.T, b1[None, :], Wsrc.T, bsrc[None, :], Wdst.T, bdst[None, :],
        Watt.T, batt[None, :], belta)
    ft = _edge_aggregate(src, dst, edge_dist, semb, demb, fsrc)
    ft2 = ft.reshape(_NC, _NPAD, _OUT_F)
    return _finalize(eatt, ft2)


# speedup vs baseline: 1.2966x; 1.2966x over previous
import functools

import jax
import jax.numpy as jnp
from jax import lax
from jax.experimental import pallas as pl
from jax.experimental.pallas import tpu as pltpu
from jax.experimental.pallas import tpu_sc as plsc

_N = 10000
_E = 320000
_IN_F = 128
_OUT_F = 128
_EMB = 32

_NC = 2
_NS = 16
_L = 16
_NW = _NC * _NS
_EPW = _E // _NW
_KC = 80
_NCHUNK = _EPW // _KC
_NPAD = 10240
_RPT = _NPAD // _NS

_ROW_BLK = 1000


def _proj_body(belta_ref, feat_ref, w1t_ref, b1_ref, wst_ref, bs_ref,
               wdt_ref, bd_ref, wat_ref, ba_ref,
               fsrc_ref, semb_ref, demb_ref, eatt_ref):
    f = feat_ref[...]
    b = belta_ref[0]
    fsrc_ref[...] = jnp.dot(f, w1t_ref[...],
                            preferred_element_type=jnp.float32) + b1_ref[...]
    semb_ref[...] = (jnp.dot(f, wst_ref[...],
                             preferred_element_type=jnp.float32)
                     + bs_ref[...]) * b
    demb_ref[...] = jnp.dot(f, wdt_ref[...],
                            preferred_element_type=jnp.float32) + bd_ref[...]
    eatt_ref[...] = jnp.dot(jnp.maximum(f, 0.0), wat_ref[...],
                            preferred_element_type=jnp.float32) + ba_ref[...]


def _projections(feat, w1t, b1, wst, bs, wdt, bd, wat, ba, belta):
    nblk = _N // _ROW_BLK
    full = lambda *_: (0, 0)
    row = lambda i: (i, 0)
    return pl.pallas_call(
        _proj_body,
        grid=(nblk,),
        in_specs=[
            pl.BlockSpec(memory_space=pltpu.SMEM),
            pl.BlockSpec((_ROW_BLK, _IN_F), row),
            pl.BlockSpec((_IN_F, _OUT_F), full),
            pl.BlockSpec((1, _OUT_F), full),
            pl.BlockSpec((_IN_F, _EMB), full),
            pl.BlockSpec((1, _EMB), full),
            pl.BlockSpec((_IN_F, _EMB), full),
            pl.BlockSpec((1, _EMB), full),
            pl.BlockSpec((_IN_F, _OUT_F), full),
            pl.BlockSpec((1, _OUT_F), full),
        ],
        out_specs=[
            pl.BlockSpec((_ROW_BLK, _OUT_F), row),
            pl.BlockSpec((_ROW_BLK, _EMB), row),
            pl.BlockSpec((_ROW_BLK, _EMB), row),
            pl.BlockSpec((_ROW_BLK, _OUT_F), row),
        ],
        out_shape=[
            jax.ShapeDtypeStruct((_N, _OUT_F), jnp.float32),
            jax.ShapeDtypeStruct((_N, _EMB), jnp.float32),
            jax.ShapeDtypeStruct((_N, _EMB), jnp.float32),
            jax.ShapeDtypeStruct((_N, _OUT_F), jnp.float32),
        ],
    )(belta, feat, w1t, b1, wst, bs, wdt, bd, wat, ba)


def _edge_body(src_hbm, dst_hbm, dist_hbm, semb_hbm, demb_hbm, fsrc_hbm,
               zeros_hbm, out_hbm,
               si0, si1, di0, di1, dv0, dv1, sr0, sr1, dr0, dr1, fr0, fr1,
               ft_sh, sem_l0, sem_l1, sem_b0, sem_b1):
    cid = lax.axis_index("c")
    sid = lax.axis_index("s")
    wid = sid * _NC + cid
    last = _NCHUNK - 1

    def fire_lin(c, si, di, dv, sem):
        pltpu.async_copy(src_hbm.at[wid, c], si, sem)
        pltpu.async_copy(dst_hbm.at[wid, c], di, sem)
        pltpu.async_copy(dist_hbm.at[wid, c], dv, sem)

    def wait_lin(si, di, dv, sem):
        pltpu.make_async_copy(src_hbm.at[wid, 0], si, sem).wait()
        pltpu.make_async_copy(dst_hbm.at[wid, 0], di, sem).wait()
        pltpu.make_async_copy(dist_hbm.at[wid, 0], dv, sem).wait()

    def fire_gath(si, di, sr, dr, fr, sem):
        pltpu.async_copy(semb_hbm.at[si], sr, sem)
        pltpu.async_copy(demb_hbm.at[di], dr, sem)
        pltpu.async_copy(fsrc_hbm.at[si], fr, sem)

    def wait_gath(si, di, sr, dr, fr, sem):
        pltpu.make_async_copy(semb_hbm.at[si], sr, sem).wait()
        pltpu.make_async_copy(demb_hbm.at[di], dr, sem).wait()
        pltpu.make_async_copy(fsrc_hbm.at[si], fr, sem).wait()

    bfly = [lax.iota(jnp.int32, _L) ^ (1 << s) for s in range(4)]

    def _allsum(v):
        for idx in bfly:
            v = v + jnp.take_along_axis(v, idx, axis=0)
        return v

    def compute_scale(dv, sr, dr, fr):
        for g in range(_KC // _L):
            invd = 1.0 / dv[pl.ds(g * _L, _L)]
            for i in range(_L):
                e = g * _L + i
                s0 = sr[e, pl.ds(0, _L)]
                s1 = sr[e, pl.ds(_L, _L)]
                d0 = dr[e, pl.ds(0, _L)]
                d1 = dr[e, pl.ds(_L, _L)]
                xe = _allsum(s0 * d0 + s1 * d1)
                w = xe * jnp.take_along_axis(
                    invd, jnp.full((_L,), i, jnp.int32), axis=0)
                for j in range(_OUT_F // _L):
                    sl = pl.ds(j * _L, _L)
                    fr[e, sl] = fr[e, sl] * w

    def scatter(di, fr):
        pltpu.sync_copy(fr, ft_sh.at[di], add=True)

    fire_lin(0, si0, di0, dv0, sem_l0)
    fire_lin(1, si1, di1, dv1, sem_l1)
    pltpu.sync_copy(zeros_hbm, ft_sh.at[pl.ds(sid * _RPT, _RPT)])
    plsc.subcore_barrier()
    wait_lin(si0, di0, dv0, sem_l0)
    fire_gath(si0, di0, sr0, dr0, fr0, sem_b0)

    def pair(p, carry):
        c0 = 2 * p
        wait_gath(si0, di0, sr0, dr0, fr0, sem_b0)
        wait_lin(si1, di1, dv1, sem_l1)
        fire_gath(si1, di1, sr1, dr1, fr1, sem_b1)
        compute_scale(dv0, sr0, dr0, fr0)
        scatter(di0, fr0)
        fire_lin(jnp.minimum(c0 + 2, last), si0, di0, dv0, sem_l0)
        wait_gath(si1, di1, sr1, dr1, fr1, sem_b1)
        wait_lin(si0, di0, dv0, sem_l0)
        fire_gath(si0, di0, sr0, dr0, fr0, sem_b0)
        compute_scale(dv1, sr1, dr1, fr1)
        scatter(di1, fr1)
        fire_lin(jnp.minimum(c0 + 3, last), si1, di1, dv1, sem_l1)
        return carry

    lax.fori_loop(0, _NCHUNK // 2, pair, 0)

    wait_gath(si0, di0, sr0, dr0, fr0, sem_b0)
    compute_scale(dv0, sr0, dr0, fr0)
    scatter(di0, fr0)
    wait_lin(si1, di1, dv1, sem_l1)

    plsc.subcore_barrier()
    off = (cid * _NS + sid) * _RPT
    pltpu.sync_copy(ft_sh.at[pl.ds(sid * _RPT, _RPT)],
                    out_hbm.at[pl.ds(off, _RPT)])


def _edge_aggregate(src, dst, dist, semb, demb, fsrc):
    src3 = src.reshape(_NW, _NCHUNK, _KC)
    dst3 = dst.reshape(_NW, _NCHUNK, _KC)
    dist3 = dist.reshape(_NW, _NCHUNK, _KC)
    zeros = jnp.zeros((_RPT, _OUT_F), jnp.float32)
    mesh = plsc.VectorSubcoreMesh(core_axis_name="c", subcore_axis_name="s")
    run = pl.kernel(
        _edge_body,
        out_type=jax.ShapeDtypeStruct((_NC * _NPAD, _OUT_F), jnp.float32),
        mesh=mesh,
        compiler_params=pltpu.CompilerParams(needs_layout_passes=False,
                                             use_tc_tiling_on_sc=False),
        scratch_types=[
            pltpu.VMEM((_KC,), jnp.int32),
            pltpu.VMEM((_KC,), jnp.int32),
            pltpu.VMEM((_KC,), jnp.int32),
            pltpu.VMEM((_KC,), jnp.int32),
            pltpu.VMEM((_KC,), jnp.float32),
            pltpu.VMEM((_KC,), jnp.float32),
            pltpu.VMEM((_KC, _EMB), jnp.float32),
            pltpu.VMEM((_KC, _EMB), jnp.float32),
            pltpu.VMEM((_KC, _EMB), jnp.float32),
            pltpu.VMEM((_KC, _EMB), jnp.float32),
            pltpu.VMEM((_KC, _OUT_F), jnp.float32),
            pltpu.VMEM((_KC, _OUT_F), jnp.float32),
            pltpu.VMEM_SHARED((_NPAD, _OUT_F), jnp.float32),
            pltpu.SemaphoreType.DMA,
            pltpu.SemaphoreType.DMA,
            pltpu.SemaphoreType.DMA,
            pltpu.SemaphoreType.DMA,
        ],
    )
    return run(src3, dst3, dist3, semb, demb, fsrc, zeros)


def _final_body(eatt_ref, ft_ref, out_ref):
    x = eatt_ref[...] * (ft_ref[0] + ft_ref[1])
    out_ref[...] = jnp.where(x > 0.0, x, jnp.exp(x) - 1.0)


def _finalize(eatt, ft2):
    nblk = _N // _ROW_BLK
    return pl.pallas_call(
        _final_body,
        grid=(nblk,),
        in_specs=[
            pl.BlockSpec((_ROW_BLK, _OUT_F), lambda i: (i, 0)),
            pl.BlockSpec((2, _ROW_BLK, _OUT_F), lambda i: (0, i, 0)),
        ],
        out_specs=pl.BlockSpec((_ROW_BLK, _OUT_F), lambda i: (i, 0)),
        out_shape=jax.ShapeDtypeStruct((_N, _OUT_F), jnp.float32),
    )(eatt, ft2)


def kernel(feat, edge_index, edge_dist, W1, b1, Wsrc, bsrc, Wdst, bdst,
           Watt, batt, belta):
    src = edge_index[0]
    dst = edge_index[1]
    fsrc, semb, demb, eatt = _projections(
        feat, W1.T, b1[None, :], Wsrc.T, bsrc[None, :], Wdst.T, bdst[None, :],
        Watt.T, batt[None, :], belta)
    ft = _edge_aggregate(src, dst, edge_dist, semb, demb, fsrc)
    ft2 = ft.reshape(_NC, _NPAD, _OUT_F)
    return _finalize(eatt, ft2)


# speedup vs baseline: 1.7836x; 1.3756x over previous
import functools

import jax
import jax.numpy as jnp
from jax import lax
from jax.experimental import pallas as pl
from jax.experimental.pallas import tpu as pltpu
from jax.experimental.pallas import tpu_sc as plsc

_N = 10000
_E = 320000
_IN_F = 128
_OUT_F = 128
_EMB = 32

_NC = 2
_NS = 16
_L = 16
_NW = _NC * _NS
_EPW = _E // _NW
_KC = 80
_NCHUNK = _EPW // _KC
_NPAD = 10240
_RPT = _NPAD // _NS

_ROW_BLK = 1000


def _proj_body(belta_ref, feat_ref, w1t_ref, b1_ref, wst_ref, bs_ref,
               wdt_ref, bd_ref, wat_ref, ba_ref,
               fsrc_ref, semb_ref, demb_ref, eatt_ref):
    f = feat_ref[...]
    b = belta_ref[0]
    fsrc_ref[...] = jnp.dot(f, w1t_ref[...],
                            preferred_element_type=jnp.float32) + b1_ref[...]
    semb_ref[...] = (jnp.dot(f, wst_ref[...],
                             preferred_element_type=jnp.float32)
                     + bs_ref[...]) * b
    demb_ref[...] = jnp.dot(f, wdt_ref[...],
                            preferred_element_type=jnp.float32) + bd_ref[...]
    eatt_ref[...] = jnp.dot(jnp.maximum(f, 0.0), wat_ref[...],
                            preferred_element_type=jnp.float32) + ba_ref[...]


def _projections(feat, w1t, b1, wst, bs, wdt, bd, wat, ba, belta):
    nblk = _N // _ROW_BLK
    full = lambda *_: (0, 0)
    row = lambda i: (i, 0)
    return pl.pallas_call(
        _proj_body,
        grid=(nblk,),
        in_specs=[
            pl.BlockSpec(memory_space=pltpu.SMEM),
            pl.BlockSpec((_ROW_BLK, _IN_F), row),
            pl.BlockSpec((_IN_F, _OUT_F), full),
            pl.BlockSpec((1, _OUT_F), full),
            pl.BlockSpec((_IN_F, _EMB), full),
            pl.BlockSpec((1, _EMB), full),
            pl.BlockSpec((_IN_F, _EMB), full),
            pl.BlockSpec((1, _EMB), full),
            pl.BlockSpec((_IN_F, _OUT_F), full),
            pl.BlockSpec((1, _OUT_F), full),
        ],
        out_specs=[
            pl.BlockSpec((_ROW_BLK, _OUT_F), row),
            pl.BlockSpec((_ROW_BLK, _EMB), row),
            pl.BlockSpec((_ROW_BLK, _EMB), row),
            pl.BlockSpec((_ROW_BLK, _OUT_F), row),
        ],
        out_shape=[
            jax.ShapeDtypeStruct((_N, _OUT_F), jnp.float32),
            jax.ShapeDtypeStruct((_N, _EMB), jnp.float32),
            jax.ShapeDtypeStruct((_N, _EMB), jnp.float32),
            jax.ShapeDtypeStruct((_N, _OUT_F), jnp.float32),
        ],
    )(belta, feat, w1t, b1, wst, bs, wdt, bd, wat, ba)


def _edge_body(src_hbm, dst_hbm, dist_hbm, semb_hbm, demb_hbm, fsrc_hbm,
               zeros_hbm, out_hbm,
               si0, si1, di0, di1, dv0, dv1, sr0, sr1, dr0, dr1, fr0, fr1,
               ds0, ds1, ft_sh, sem_l0, sem_l1, sem_b0, sem_b1,
               sem_s0, sem_s1):
    cid = lax.axis_index("c")
    sid = lax.axis_index("s")
    wid = sid * _NC + cid
    last = _NCHUNK - 1

    def fire_lin(c, si, di, dv, sem):
        pltpu.async_copy(src_hbm.at[wid, c], si, sem)
        pltpu.async_copy(dst_hbm.at[wid, c], di, sem)
        pltpu.async_copy(dist_hbm.at[wid, c], dv, sem)

    def wait_lin(si, di, dv, sem):
        pltpu.make_async_copy(src_hbm.at[wid, 0], si, sem).wait()
        pltpu.make_async_copy(dst_hbm.at[wid, 0], di, sem).wait()
        pltpu.make_async_copy(dist_hbm.at[wid, 0], dv, sem).wait()

    def fire_gath(si, di, sr, dr, fr, sem):
        pltpu.async_copy(semb_hbm.at[si], sr, sem)
        pltpu.async_copy(demb_hbm.at[di], dr, sem)
        pltpu.async_copy(fsrc_hbm.at[si], fr, sem)

    def wait_gath(si, di, sr, dr, fr, sem):
        pltpu.make_async_copy(semb_hbm.at[si], sr, sem).wait()
        pltpu.make_async_copy(demb_hbm.at[di], dr, sem).wait()
        pltpu.make_async_copy(fsrc_hbm.at[si], fr, sem).wait()

    def compute_scale(dv, sr, dr, fr):
        for g in range(_KC // _L):
            invd = 1.0 / dv[pl.ds(g * _L, _L)]
            for i in range(_L):
                e = g * _L + i
                s0 = sr[e, pl.ds(0, _L)]
                s1 = sr[e, pl.ds(_L, _L)]
                d0 = dr[e, pl.ds(0, _L)]
                d1 = dr[e, pl.ds(_L, _L)]
                xe = jnp.sum(s0 * d0 + s1 * d1)
                w = jnp.broadcast_to(xe, (_L,)) * jnp.broadcast_to(invd[i], (_L,))
                for j in range(_OUT_F // _L):
                    sl = pl.ds(j * _L, _L)
                    fr[e, sl] = fr[e, sl] * w

    def copy_didx(di, dscat):
        for k in range(_KC // _L):
            sl = pl.ds(k * _L, _L)
            dscat[sl] = di[sl]

    def fire_scat(dscat, fr, sem):
        pltpu.async_copy(fr, ft_sh.at[dscat], sem, add=True)

    def wait_scat(dscat, fr, sem):
        pltpu.make_async_copy(fr, ft_sh.at[dscat], sem).wait()

    fire_lin(0, si0, di0, dv0, sem_l0)
    fire_lin(1, si1, di1, dv1, sem_l1)
    pltpu.sync_copy(zeros_hbm, ft_sh.at[pl.ds(sid * _RPT, _RPT)])
    plsc.subcore_barrier()
    wait_lin(si0, di0, dv0, sem_l0)
    fire_gath(si0, di0, sr0, dr0, fr0, sem_b0)

    wait_gath(si0, di0, sr0, dr0, fr0, sem_b0)
    copy_didx(di0, ds0)
    wait_lin(si1, di1, dv1, sem_l1)
    fire_gath(si1, di1, sr1, dr1, fr1, sem_b1)
    compute_scale(dv0, sr0, dr0, fr0)
    fire_scat(ds0, fr0, sem_s0)
    fire_lin(2, si0, di0, dv0, sem_l0)

    def pair(p, carry):
        c0 = 2 * p + 1
        wait_gath(si1, di1, sr1, dr1, fr1, sem_b1)
        copy_didx(di1, ds1)
        wait_lin(si0, di0, dv0, sem_l0)
        wait_scat(ds0, fr0, sem_s0)
        fire_gath(si0, di0, sr0, dr0, fr0, sem_b0)
        compute_scale(dv1, sr1, dr1, fr1)
        fire_scat(ds1, fr1, sem_s1)
        fire_lin(jnp.minimum(c0 + 2, last), si1, di1, dv1, sem_l1)
        wait_gath(si0, di0, sr0, dr0, fr0, sem_b0)
        copy_didx(di0, ds0)
        wait_lin(si1, di1, dv1, sem_l1)
        wait_scat(ds1, fr1, sem_s1)
        fire_gath(si1, di1, sr1, dr1, fr1, sem_b1)
        compute_scale(dv0, sr0, dr0, fr0)
        fire_scat(ds0, fr0, sem_s0)
        fire_lin(jnp.minimum(c0 + 3, last), si0, di0, dv0, sem_l0)
        return carry

    lax.fori_loop(0, (_NCHUNK - 1) // 2, pair, 0)

    wait_gath(si1, di1, sr1, dr1, fr1, sem_b1)
    wait_lin(si0, di0, dv0, sem_l0)
    wait_scat(ds0, fr0, sem_s0)

    plsc.subcore_barrier()
    off = (cid * _NS + sid) * _RPT
    pltpu.sync_copy(ft_sh.at[pl.ds(sid * _RPT, _RPT)],
                    out_hbm.at[pl.ds(off, _RPT)])


def _edge_aggregate(src, dst, dist, semb, demb, fsrc):
    src3 = src.reshape(_NW, _NCHUNK, _KC)
    dst3 = dst.reshape(_NW, _NCHUNK, _KC)
    dist3 = dist.reshape(_NW, _NCHUNK, _KC)
    zeros = jnp.zeros((_RPT, _OUT_F), jnp.float32)
    mesh = plsc.VectorSubcoreMesh(core_axis_name="c", subcore_axis_name="s")
    run = pl.kernel(
        _edge_body,
        out_type=jax.ShapeDtypeStruct((_NC * _NPAD, _OUT_F), jnp.float32),
        mesh=mesh,
        compiler_params=pltpu.CompilerParams(needs_layout_passes=False,
                                             use_tc_tiling_on_sc=False),
        scratch_types=[
            pltpu.VMEM((_KC,), jnp.int32),
            pltpu.VMEM((_KC,), jnp.int32),
            pltpu.VMEM((_KC,), jnp.int32),
            pltpu.VMEM((_KC,), jnp.int32),
            pltpu.VMEM((_KC,), jnp.float32),
            pltpu.VMEM((_KC,), jnp.float32),
            pltpu.VMEM((_KC, _EMB), jnp.float32),
            pltpu.VMEM((_KC, _EMB), jnp.float32),
            pltpu.VMEM((_KC, _EMB), jnp.float32),
            pltpu.VMEM((_KC, _EMB), jnp.float32),
            pltpu.VMEM((_KC, _OUT_F), jnp.float32),
            pltpu.VMEM((_KC, _OUT_F), jnp.float32),
            pltpu.VMEM((_KC,), jnp.int32),
            pltpu.VMEM((_KC,), jnp.int32),
            pltpu.VMEM_SHARED((_NPAD, _OUT_F), jnp.float32),
            pltpu.SemaphoreType.DMA,
            pltpu.SemaphoreType.DMA,
            pltpu.SemaphoreType.DMA,
            pltpu.SemaphoreType.DMA,
            pltpu.SemaphoreType.DMA,
            pltpu.SemaphoreType.DMA,
        ],
    )
    return run(src3, dst3, dist3, semb, demb, fsrc, zeros)


def _final_body(eatt_ref, ft_ref, out_ref):
    x = eatt_ref[...] * (ft_ref[0] + ft_ref[1])
    out_ref[...] = jnp.where(x > 0.0, x, jnp.exp(x) - 1.0)


def _finalize(eatt, ft2):
    nblk = _N // _ROW_BLK
    return pl.pallas_call(
        _final_body,
        grid=(nblk,),
        in_specs=[
            pl.BlockSpec((_ROW_BLK, _OUT_F), lambda i: (i, 0)),
            pl.BlockSpec((2, _ROW_BLK, _OUT_F), lambda i: (0, i, 0)),
        ],
        out_specs=pl.BlockSpec((_ROW_BLK, _OUT_F), lambda i: (i, 0)),
        out_shape=jax.ShapeDtypeStruct((_N, _OUT_F), jnp.float32),
    )(eatt, ft2)


def kernel(feat, edge_index, edge_dist, W1, b1, Wsrc, bsrc, Wdst, bdst,
           Watt, batt, belta):
    src = edge_index[0]
    dst = edge_index[1]
    fsrc, semb, demb, eatt = _projections(
        feat, W1.T, b1[None, :], Wsrc.T, bsrc[None, :], Wdst.T, bdst[None, :],
        Watt.T, batt[None, :], belta)
    ft = _edge_aggregate(src, dst, edge_dist, semb, demb, fsrc)
    ft2 = ft.reshape(_NC, _NPAD, _OUT_F)
    return _finalize(eatt, ft2)


# speedup vs baseline: 2.6662x; 1.4948x over previous
import functools

import jax
import jax.numpy as jnp
from jax import lax
from jax.experimental import pallas as pl
from jax.experimental.pallas import tpu as pltpu
from jax.experimental.pallas import tpu_sc as plsc

_N = 10000
_E = 320000
_IN_F = 128
_OUT_F = 128
_EMB = 32

_NC = 2
_NS = 16
_L = 16
_NW = _NC * _NS
_EPW = _E // _NW
_KC = 80
_NCHUNK = _EPW // _KC
_NPAD = 10240
_RPT = _NPAD // _NS

_ROW_BLK = 1000


def _proj_body(belta_ref, feat_ref, w1t_ref, b1_ref, wst_ref, bs_ref,
               wdt_ref, bd_ref, wat_ref, ba_ref,
               fsrc_ref, semb_ref, demb_ref, eatt_ref):
    f = feat_ref[...]
    b = belta_ref[0]
    fsrc_ref[...] = (jnp.dot(f, w1t_ref[...],
                             preferred_element_type=jnp.float32)
                     + b1_ref[...]).astype(jnp.bfloat16)
    semb_ref[...] = (jnp.dot(f, wst_ref[...],
                             preferred_element_type=jnp.float32)
                     + bs_ref[...]) * b
    demb_ref[...] = jnp.dot(f, wdt_ref[...],
                            preferred_element_type=jnp.float32) + bd_ref[...]
    eatt_ref[...] = jnp.dot(jnp.maximum(f, 0.0), wat_ref[...],
                            preferred_element_type=jnp.float32) + ba_ref[...]


def _projections(feat, w1t, b1, wst, bs, wdt, bd, wat, ba, belta):
    nblk = _N // _ROW_BLK
    full = lambda *_: (0, 0)
    row = lambda i: (i, 0)
    return pl.pallas_call(
        _proj_body,
        grid=(nblk,),
        in_specs=[
            pl.BlockSpec(memory_space=pltpu.SMEM),
            pl.BlockSpec((_ROW_BLK, _IN_F), row),
            pl.BlockSpec((_IN_F, _OUT_F), full),
            pl.BlockSpec((1, _OUT_F), full),
            pl.BlockSpec((_IN_F, _EMB), full),
            pl.BlockSpec((1, _EMB), full),
            pl.BlockSpec((_IN_F, _EMB), full),
            pl.BlockSpec((1, _EMB), full),
            pl.BlockSpec((_IN_F, _OUT_F), full),
            pl.BlockSpec((1, _OUT_F), full),
        ],
        out_specs=[
            pl.BlockSpec((_ROW_BLK, _OUT_F), row),
            pl.BlockSpec((_ROW_BLK, _EMB), row),
            pl.BlockSpec((_ROW_BLK, _EMB), row),
            pl.BlockSpec((_ROW_BLK, _OUT_F), row),
        ],
        out_shape=[
            jax.ShapeDtypeStruct((_N, _OUT_F), jnp.bfloat16),
            jax.ShapeDtypeStruct((_N, _EMB), jnp.float32),
            jax.ShapeDtypeStruct((_N, _EMB), jnp.float32),
            jax.ShapeDtypeStruct((_N, _OUT_F), jnp.float32),
        ],
    )(belta, feat, w1t, b1, wst, bs, wdt, bd, wat, ba)


def _edge_body(src_hbm, dst_hbm, dist_hbm, semb_hbm, demb_hbm, fsrc_hbm,
               zeros_hbm, out_hbm,
               si0, si1, di0, di1, dv0, dv1, sr0, sr1, dr0, dr1, fb0, fb1,
               ms0, ms1, ds0, ds1, ft_sh, sem_l0, sem_l1, sem_b0, sem_b1,
               sem_s0, sem_s1):
    cid = lax.axis_index("c")
    sid = lax.axis_index("s")
    wid = sid * _NC + cid
    last = _NCHUNK - 1

    def fire_lin(c, si, di, dv, sem):
        pltpu.async_copy(src_hbm.at[wid, c], si, sem)
        pltpu.async_copy(dst_hbm.at[wid, c], di, sem)
        pltpu.async_copy(dist_hbm.at[wid, c], dv, sem)

    def wait_lin(si, di, dv, sem):
        pltpu.make_async_copy(src_hbm.at[wid, 0], si, sem).wait()
        pltpu.make_async_copy(dst_hbm.at[wid, 0], di, sem).wait()
        pltpu.make_async_copy(dist_hbm.at[wid, 0], dv, sem).wait()

    def fire_gath(si, di, sr, dr, fb, sem):
        pltpu.async_copy(semb_hbm.at[si], sr, sem)
        pltpu.async_copy(demb_hbm.at[di], dr, sem)
        pltpu.async_copy(fsrc_hbm.at[si], fb, sem)

    def wait_gath(si, di, sr, dr, fb, sem):
        pltpu.make_async_copy(semb_hbm.at[si], sr, sem).wait()
        pltpu.make_async_copy(demb_hbm.at[di], dr, sem).wait()
        pltpu.make_async_copy(fsrc_hbm.at[si], fb, sem).wait()

    def compute_scale(dv, sr, dr, fb, ms):
        for g in range(_KC // _L):
            invd = 1.0 / dv[pl.ds(g * _L, _L)]
            for i in range(_L):
                e = g * _L + i
                s0 = sr[e, pl.ds(0, _L)]
                s1 = sr[e, pl.ds(_L, _L)]
                d0 = dr[e, pl.ds(0, _L)]
                d1 = dr[e, pl.ds(_L, _L)]
                xe = jnp.sum(s0 * d0 + s1 * d1)
                w = jnp.broadcast_to(xe, (_L,)) * jnp.broadcast_to(invd[i], (_L,))
                for t in range(_OUT_F // (2 * _L)):
                    v = fb[e, pl.ds(2 * _L * t, 2 * _L)]
                    a, b = plsc.unpack(v, format=plsc.PackFormat.INTERLEAVED)
                    ms[e, pl.ds(2 * _L * t, _L)] = a * w
                    ms[e, pl.ds(2 * _L * t + _L, _L)] = b * w

    def copy_didx(di, dscat):
        for k in range(_KC // _L):
            sl = pl.ds(k * _L, _L)
            dscat[sl] = di[sl]

    def fire_scat(dscat, ms, sem):
        pltpu.async_copy(ms, ft_sh.at[dscat], sem, add=True)

    def wait_scat(dscat, ms, sem):
        pltpu.make_async_copy(ms, ft_sh.at[dscat], sem).wait()

    fire_lin(0, si0, di0, dv0, sem_l0)
    fire_lin(1, si1, di1, dv1, sem_l1)
    pltpu.sync_copy(zeros_hbm, ft_sh.at[pl.ds(sid * _RPT, _RPT)])
    plsc.subcore_barrier()
    wait_lin(si0, di0, dv0, sem_l0)
    fire_gath(si0, di0, sr0, dr0, fb0, sem_b0)

    wait_gath(si0, di0, sr0, dr0, fb0, sem_b0)
    copy_didx(di0, ds0)
    wait_lin(si1, di1, dv1, sem_l1)
    fire_gath(si1, di1, sr1, dr1, fb1, sem_b1)
    compute_scale(dv0, sr0, dr0, fb0, ms0)
    fire_scat(ds0, ms0, sem_s0)
    fire_lin(2, si0, di0, dv0, sem_l0)

    def pair(p, carry):
        c0 = 2 * p + 1
        wait_gath(si1, di1, sr1, dr1, fb1, sem_b1)
        copy_didx(di1, ds1)
        wait_lin(si0, di0, dv0, sem_l0)
        wait_scat(ds0, ms0, sem_s0)
        fire_gath(si0, di0, sr0, dr0, fb0, sem_b0)
        compute_scale(dv1, sr1, dr1, fb1, ms1)
        fire_scat(ds1, ms1, sem_s1)
        fire_lin(jnp.minimum(c0 + 2, last), si1, di1, dv1, sem_l1)
        wait_gath(si0, di0, sr0, dr0, fb0, sem_b0)
        copy_didx(di0, ds0)
        wait_lin(si1, di1, dv1, sem_l1)
        wait_scat(ds1, ms1, sem_s1)
        fire_gath(si1, di1, sr1, dr1, fb1, sem_b1)
        compute_scale(dv0, sr0, dr0, fb0, ms0)
        fire_scat(ds0, ms0, sem_s0)
        fire_lin(jnp.minimum(c0 + 3, last), si0, di0, dv0, sem_l0)
        return carry

    lax.fori_loop(0, (_NCHUNK - 1) // 2, pair, 0)

    wait_gath(si1, di1, sr1, dr1, fb1, sem_b1)
    wait_lin(si0, di0, dv0, sem_l0)
    wait_scat(ds0, ms0, sem_s0)

    plsc.subcore_barrier()
    off = (cid * _NS + sid) * _RPT
    pltpu.sync_copy(ft_sh.at[pl.ds(sid * _RPT, _RPT)],
                    out_hbm.at[pl.ds(off, _RPT)])


def _edge_aggregate(src, dst, dist, semb, demb, fsrc):
    src3 = src.reshape(_NW, _NCHUNK, _KC)
    dst3 = dst.reshape(_NW, _NCHUNK, _KC)
    dist3 = dist.reshape(_NW, _NCHUNK, _KC)
    zeros = jnp.zeros((_RPT, _OUT_F), jnp.float32)
    mesh = plsc.VectorSubcoreMesh(core_axis_name="c", subcore_axis_name="s")
    run = pl.kernel(
        _edge_body,
        out_type=jax.ShapeDtypeStruct((_NC * _NPAD, _OUT_F), jnp.float32),
        mesh=mesh,
        compiler_params=pltpu.CompilerParams(needs_layout_passes=False,
                                             use_tc_tiling_on_sc=False),
        scratch_types=[
            pltpu.VMEM((_KC,), jnp.int32),
            pltpu.VMEM((_KC,), jnp.int32),
            pltpu.VMEM((_KC,), jnp.int32),
            pltpu.VMEM((_KC,), jnp.int32),
            pltpu.VMEM((_KC,), jnp.float32),
            pltpu.VMEM((_KC,), jnp.float32),
            pltpu.VMEM((_KC, _EMB), jnp.float32),
            pltpu.VMEM((_KC, _EMB), jnp.float32),
            pltpu.VMEM((_KC, _EMB), jnp.float32),
            pltpu.VMEM((_KC, _EMB), jnp.float32),
            pltpu.VMEM((_KC, _OUT_F), jnp.bfloat16),
            pltpu.VMEM((_KC, _OUT_F), jnp.bfloat16),
            pltpu.VMEM((_KC, _OUT_F), jnp.float32),
            pltpu.VMEM((_KC, _OUT_F), jnp.float32),
            pltpu.VMEM((_KC,), jnp.int32),
            pltpu.VMEM((_KC,), jnp.int32),
            pltpu.VMEM_SHARED((_NPAD, _OUT_F), jnp.float32),
            pltpu.SemaphoreType.DMA,
            pltpu.SemaphoreType.DMA,
            pltpu.SemaphoreType.DMA,
            pltpu.SemaphoreType.DMA,
            pltpu.SemaphoreType.DMA,
            pltpu.SemaphoreType.DMA,
        ],
    )
    return run(src3, dst3, dist3, semb, demb, fsrc, zeros)


def _final_body(eatt_ref, ft_ref, out_ref):
    x = eatt_ref[...] * (ft_ref[0] + ft_ref[1])
    out_ref[...] = jnp.where(x > 0.0, x, jnp.exp(x) - 1.0)


def _finalize(eatt, ft2):
    nblk = _N // _ROW_BLK
    return pl.pallas_call(
        _final_body,
        grid=(nblk,),
        in_specs=[
            pl.BlockSpec((_ROW_BLK, _OUT_F), lambda i: (i, 0)),
            pl.BlockSpec((2, _ROW_BLK, _OUT_F), lambda i: (0, i, 0)),
        ],
        out_specs=pl.BlockSpec((_ROW_BLK, _OUT_F), lambda i: (i, 0)),
        out_shape=jax.ShapeDtypeStruct((_N, _OUT_F), jnp.float32),
    )(eatt, ft2)


_PERM = tuple(
    32 * t + (j // 2 if j % 2 == 0 else 16 + j // 2)
    for t in range(4) for j in range(32)
)


def kernel(feat, edge_index, edge_dist, W1, b1, Wsrc, bsrc, Wdst, bdst,
           Watt, batt, belta):
    src = edge_index[0]
    dst = edge_index[1]
    perm = jnp.array(_PERM, jnp.int32)
    fsrc, semb, demb, eatt = _projections(
        feat, W1.T[:, perm], b1[perm][None, :], Wsrc.T, bsrc[None, :],
        Wdst.T, bdst[None, :], Watt.T, batt[None, :], belta)
    ft = _edge_aggregate(src, dst, edge_dist, semb, demb, fsrc)
    ft2 = ft.reshape(_NC, _NPAD, _OUT_F)
    return _finalize(eatt, ft2)
